# baseline (speedup 1.0000x reference)
import functools

import jax
import jax.numpy as jnp
from jax import lax
from jax.experimental import pallas as pl
from jax.experimental.pallas import tpu as pltpu
from jax.experimental.pallas import tpu_sc as plsc

R, N, L = 32, 32768, 16
NB = 256
NBP = NB + 1
NC, NS = 2, 16
MINT32 = -2**31
HIST_WORDS = 4224


def _sortable_key(v):
    b = plsc.bitcast(v, jnp.int32)
    m = (b >> 31) | jnp.int32(MINT32)
    return b ^ m


def _suffix_pick(v, k):
    sfx = lax.rev(plsc.cumsum(lax.rev(v, (0,))), (0,))
    m = sfx >= k
    cnt = plsc.all_reduce_population_count(m)[0]
    idx = cnt - 1
    onehot = lax.iota(jnp.int32, 16) == idx
    val = jnp.sum(jnp.where(onehot, v, 0))
    sfx_i = jnp.sum(jnp.where(onehot, sfx, 0))
    kp = k - (sfx_i - val)
    return idx, kp, val


NCHUNK = 8
CW = N // NCHUNK


def _sc_body(base, scores_hbm, kvec_hbm, out_hbm, row_v, hist_v, total_v,
             kv_v, c1_v, c2_v, *sems):
    wid = base + lax.axis_index("s")
    in_copies = [
        pltpu.async_copy(scores_hbm.at[wid, pl.ds(c * CW, CW)],
                         row_v.at[pl.ds(c * CW, CW)], sems[c])
        for c in range(NCHUNK)
    ]
    pltpu.sync_copy(kvec_hbm, kv_v)
    k_rem = kv_v[...][0]

    lane = lax.iota(jnp.int32, L)
    ones = jnp.ones((L,), jnp.int32)
    zeros16 = jnp.zeros((16,), jnp.int32)
    U = 8

    def zero_hist():
        @plsc.parallel_loop(0, HIST_WORDS // 16, unroll=U)
        def _(i):
            hist_v[pl.ds(i * 16, 16)] = zeros16

    def merge_hist():
        def merge_body(c, chunks):
            vs = [hist_v[pl.ds(l * NBP + c * 16, 16)] for l in range(L)]
            while len(vs) > 1:
                vs = [a + b for a, b in zip(vs[::2], vs[1::2])]
            total_v[pl.ds(c * 16, 16)] = vs[0]
            return jnp.where(lane == c, jnp.sum(vs[0]), chunks)
        return lax.fori_loop(0, NB // 16, merge_body, zeros16)

    def select(chunks, k):
        cstar, kp, _ = _suffix_pick(chunks, k)
        v = total_v[pl.ds(cstar * 16, 16)]
        t_loc, knext, _ = _suffix_pick(v, kp)
        return cstar * 16 + t_loc, knext

    zero_hist()

    lane_off = lane * NBP

    for c in range(NCHUNK):
        in_copies[c].wait()

        @plsc.parallel_loop(c * (CW // L), (c + 1) * (CW // L), unroll=U)
        def _(i):
            key = _sortable_key(row_v[pl.ds(i * L, L)])
            bins = lax.shift_right_logical(key, 24)
            plsc.addupdate_scatter(hist_v, [lane_off + bins], ones)

    t, k_rem = select(merge_hist(), k_rem)
    prefix = t

    @plsc.parallel_loop(0, N // L, unroll=U, carry=jnp.int32(0))
    def compact0_loop(i, off):
        key = _sortable_key(row_v[pl.ds(i * L, L)])
        match = lax.shift_right_logical(key, 24) == prefix
        plsc.store_compressed(c1_v.at[pl.ds(off, L)], key, mask=match)
        return off + plsc.all_reduce_population_count(match)[0]
    m_cand = compact0_loop

    bufs = (c1_v, c2_v)
    for level in range(1, 4):
        shift = 24 - 8 * level
        src, dst = bufs[(level - 1) % 2], bufs[level % 2]
        nblk = (m_cand + (L - 1)) // L
        zero_hist()

        @plsc.parallel_loop(0, nblk, unroll=2)
        def _(i, src=src, shift=shift, m_cand=m_cand):
            key = src[pl.ds(i * L, L)]
            valid = (i * L + lane) < m_cand
            bins = lax.shift_right_logical(key, shift) & 0xFF
            plsc.addupdate_scatter(hist_v, [lane_off + bins], ones,
                                   mask=valid)

        t, k_rem = select(merge_hist(), k_rem)
        prefix = lax.shift_left(prefix, 8) | t

        if level < 3:
            @plsc.parallel_loop(0, nblk, unroll=2, carry=jnp.int32(0))
            def compactl_loop(i, off, src=src, dst=dst, shift=shift,
                              m_cand=m_cand, t=t):
                key = src[pl.ds(i * L, L)]
                valid = (i * L + lane) < m_cand
                match = jnp.logical_and(
                    valid, (lax.shift_right_logical(key, shift) & 0xFF) == t)
                plsc.store_compressed(dst.at[pl.ds(off, L)], key, mask=match)
                return off + plsc.all_reduce_population_count(match)[0]
            m_cand = compactl_loop

    bmask = jnp.where(prefix < 0, jnp.int32(MINT32), jnp.int32(-1))
    tbits = jnp.broadcast_to(prefix ^ bmask, (L,))
    thresh = plsc.bitcast(tbits, jnp.float32)

    out_copies = []
    for c in range(NCHUNK):
        @plsc.parallel_loop(c * (CW // L), (c + 1) * (CW // L), unroll=U)
        def _(i):
            v = row_v[pl.ds(i * L, L)]
            row_v[pl.ds(i * L, L)] = jnp.where(
                v >= thresh, v, jnp.float32(-1e9))

        out_copies.append(
            pltpu.async_copy(row_v.at[pl.ds(c * CW, CW)],
                             out_hbm.at[wid - base, pl.ds(c * CW, CW)],
                             sems[c]))

    for h in out_copies:
        h.wait()


def _make_half(base):
    return functools.partial(
        pl.kernel,
        out_type=jax.ShapeDtypeStruct((NS, N), jnp.float32),
        mesh=plsc.VectorSubcoreMesh(
            core_axis_name="c", subcore_axis_name="s",
            num_cores=1, num_subcores=NS),
        compiler_params=pltpu.CompilerParams(needs_layout_passes=False),
        scratch_types=[
            pltpu.VMEM((N,), jnp.float32),
            pltpu.VMEM((HIST_WORDS,), jnp.int32),
            pltpu.VMEM((NB,), jnp.int32),
            pltpu.VMEM((L,), jnp.int32),
            pltpu.VMEM((N + L,), jnp.int32),
            pltpu.VMEM((N + L,), jnp.int32),
        ] + [pltpu.SemaphoreType.DMA] * NCHUNK,
        name=f"topk_mask_rows_{base}",
    )(functools.partial(_sc_body, base))


_sc_halves = None


def kernel(scores, k):
    global _sc_halves
    if _sc_halves is None:
        _sc_halves = (_make_half(0), _make_half(NS))
    kvec = jnp.full((L,), k, jnp.int32)
    lo = _sc_halves[0](scores, kvec)
    hi = _sc_halves[1](scores, kvec)
    return jnp.concatenate([lo, hi], axis=0)


# speedup vs baseline: 1.6759x; 1.6759x over previous
import functools

import jax
import jax.numpy as jnp
from jax import lax
from jax.experimental import pallas as pl
from jax.experimental.pallas import tpu as pltpu
from jax.experimental.pallas import tpu_sc as plsc

R, N, L = 32, 32768, 16
NB = 256
NBP = NB + 1
NC, NS = 2, 16
MINT32 = -2**31
HIST_WORDS = 4224
NCHUNK = 8
CW = N // NCHUNK


def _suffix_pick(v, k):
    sfx = lax.rev(plsc.cumsum(lax.rev(v, (0,))), (0,))
    m = sfx >= k
    cnt = plsc.all_reduce_population_count(m)[0]
    idx = cnt - 1
    onehot = lax.iota(jnp.int32, 16) == idx
    val = jnp.sum(jnp.where(onehot, v, 0))
    sfx_i = jnp.sum(jnp.where(onehot, sfx, 0))
    kp = k - (sfx_i - val)
    return idx, kp, val


def _sc_body(scores_hbm, kvec_hbm, out_hbm, row_v, hist_v, total_v, kv_v,
             c1_v, c2_v, *sems):
    wid = lax.axis_index("s") * NC + lax.axis_index("c")
    in_copies = [
        pltpu.async_copy(scores_hbm.at[wid, pl.ds(c * CW, CW)],
                         row_v.at[pl.ds(c * CW, CW)], sems[c])
        for c in range(NCHUNK)
    ]
    pltpu.sync_copy(kvec_hbm, kv_v)
    k_rem = kv_v[...][0]

    lane = lax.iota(jnp.int32, L)
    ones = jnp.ones((L,), jnp.int32)
    zeros16 = jnp.zeros((16,), jnp.int32)
    lane_off = lane * NBP
    U = 8

    def zero_hist():
        @plsc.parallel_loop(0, HIST_WORDS // 16, unroll=U)
        def _(i):
            hist_v[pl.ds(i * 16, 16)] = zeros16

    def merge_hist(remap):
        def merge_body(c, chunks):
            vs = [hist_v[pl.ds(l * NBP + c * 16, 16)] for l in range(L)]
            while len(vs) > 1:
                vs = [a + b for a, b in zip(vs[::2], vs[1::2])]
            acc = vs[0]
            if remap:
                pos = c < 8
                tgt = jnp.where(pos, c + 8, 15 - c)
                acc = jnp.where(pos, acc, lax.rev(acc, (0,)))
            else:
                tgt = c
            total_v[pl.ds(tgt * 16, 16)] = acc
            return jnp.where(lane == tgt, jnp.sum(acc), chunks)
        return lax.fori_loop(0, NB // 16, merge_body, zeros16)

    def select(chunks, k):
        cstar, kp, _ = _suffix_pick(chunks, k)
        v = total_v[pl.ds(cstar * 16, 16)]
        t_loc, knext, _ = _suffix_pick(v, kp)
        return cstar * 16 + t_loc, knext

    zero_hist()

    for c in range(NCHUNK):
        in_copies[c].wait()

        @plsc.parallel_loop(c * (CW // L), (c + 1) * (CW // L), unroll=U)
        def _(i):
            b = plsc.bitcast(row_v[pl.ds(i * L, L)], jnp.int32)
            bins = lax.shift_right_logical(b, 24)
            plsc.addupdate_scatter(hist_v, [lane_off + bins], ones)

    t, k_rem = select(merge_hist(remap=True), k_rem)
    neg = t < 128
    raw_t = jnp.where(neg, 255 - t, t - 128)
    nm = jnp.where(neg, jnp.int32(0xFF), jnp.int32(0))
    prefix = raw_t

    @plsc.parallel_loop(0, N // L, unroll=U, carry=jnp.int32(0))
    def compact0_loop(i, off):
        b = plsc.bitcast(row_v[pl.ds(i * L, L)], jnp.int32)
        match = lax.shift_right_logical(b, 24) == raw_t
        plsc.store_compressed(c1_v.at[pl.ds(off, L)], b, mask=match)
        return off + plsc.all_reduce_population_count(match)[0]
    m_cand = compact0_loop

    bufs = (c1_v, c2_v)
    for level in range(1, 4):
        shift = 24 - 8 * level
        src, dst = bufs[(level - 1) % 2], bufs[level % 2]
        nblk = (m_cand + (L - 1)) // L
        zero_hist()

        @plsc.parallel_loop(0, nblk, unroll=2)
        def _(i, src=src, shift=shift, m_cand=m_cand):
            b = src[pl.ds(i * L, L)]
            valid = (i * L + lane) < m_cand
            bins = (lax.shift_right_logical(b, shift) & 0xFF) ^ nm
            plsc.addupdate_scatter(hist_v, [lane_off + bins], ones,
                                   mask=valid)

        t, k_rem = select(merge_hist(remap=False), k_rem)
        raw_b = t ^ nm
        prefix = lax.shift_left(prefix, 8) | raw_b

        if level < 3:
            @plsc.parallel_loop(0, nblk, unroll=2, carry=jnp.int32(0))
            def compactl_loop(i, off, src=src, dst=dst, shift=shift,
                              m_cand=m_cand, raw_b=raw_b):
                b = src[pl.ds(i * L, L)]
                valid = (i * L + lane) < m_cand
                match = jnp.logical_and(
                    valid,
                    (lax.shift_right_logical(b, shift) & 0xFF) == raw_b)
                plsc.store_compressed(dst.at[pl.ds(off, L)], b, mask=match)
                return off + plsc.all_reduce_population_count(match)[0]
            m_cand = compactl_loop

    thresh = plsc.bitcast(jnp.broadcast_to(prefix, (L,)), jnp.float32)

    out_copies = []
    for c in range(NCHUNK):
        @plsc.parallel_loop(c * (CW // L), (c + 1) * (CW // L), unroll=U)
        def _(i):
            v = row_v[pl.ds(i * L, L)]
            row_v[pl.ds(i * L, L)] = jnp.where(
                v >= thresh, v, jnp.float32(-1e9))

        out_copies.append(
            pltpu.async_copy(row_v.at[pl.ds(c * CW, CW)],
                             out_hbm.at[wid, pl.ds(c * CW, CW)], sems[c]))

    for h in out_copies:
        h.wait()


_sc_topk_mask = functools.partial(
    pl.kernel,
    out_type=jax.ShapeDtypeStruct((R, N), jnp.float32),
    mesh=plsc.VectorSubcoreMesh(
        core_axis_name="c", subcore_axis_name="s",
        num_cores=NC, num_subcores=NS),
    compiler_params=pltpu.CompilerParams(needs_layout_passes=False),
    scratch_types=[
        pltpu.VMEM((N,), jnp.float32),
        pltpu.VMEM((HIST_WORDS,), jnp.int32),
        pltpu.VMEM((NB,), jnp.int32),
        pltpu.VMEM((L,), jnp.int32),
        pltpu.VMEM((N + L,), jnp.int32),
        pltpu.VMEM((N + L,), jnp.int32),
    ] + [pltpu.SemaphoreType.DMA] * NCHUNK,
)(_sc_body)


def kernel(scores, k):
    kvec = jnp.full((L,), k, jnp.int32)
    return _sc_topk_mask(scores, kvec)
